# baseline (speedup 1.0000x reference)
import jax
import jax.numpy as jnp
from jax.experimental import pallas as pl
from jax.experimental.pallas import tpu as pltpu

ROWS = 1_000_000
COLS = 32


def _tc_body(x_ref, o_ref, sem):
    cp = pltpu.make_async_copy(x_ref, o_ref, sem)
    cp.start()
    cp.wait()


@jax.jit
def kernel(x):
    return pl.pallas_call(
        _tc_body,
        in_specs=[pl.BlockSpec(memory_space=pl.ANY)],
        out_specs=pl.BlockSpec(memory_space=pl.ANY),
        out_shape=jax.ShapeDtypeStruct((ROWS, COLS), jnp.float32),
        scratch_shapes=[pltpu.SemaphoreType.DMA],
    )(x)


# speedup vs baseline: 17.8138x; 17.8138x over previous
import jax
import jax.numpy as jnp
from jax.experimental import pallas as pl
from jax.experimental.pallas import tpu as pltpu

ROWS = 1_000_000
COLS = 32
BR = 8000


def _tc_body(x_ref, o_ref):
    o_ref[...] = x_ref[...]


@jax.jit
def kernel(x):
    return pl.pallas_call(
        _tc_body,
        grid=(ROWS // BR,),
        in_specs=[pl.BlockSpec((BR, COLS), lambda i: (i, 0))],
        out_specs=pl.BlockSpec((BR, COLS), lambda i: (i, 0)),
        out_shape=jax.ShapeDtypeStruct((ROWS, COLS), jnp.float32),
    )(x)
